# baseline (speedup 1.0000x reference)
import functools

import jax
import jax.numpy as jnp
from jax import lax
from jax.experimental import pallas as pl
from jax.experimental.pallas import tpu as pltpu
from jax.experimental.pallas import tpu_sc as plsc

BATCH = 16384
DIM = 64
NC, NS = 2, 16
NW = NC * NS
B_PER_W = BATCH // NW
CH = 128
NCH = B_PER_W // CH


def _sc_gather4(u2, i2, t_mfu, t_mfi, t_mlu, t_mli):
    mesh = plsc.VectorSubcoreMesh(core_axis_name="c", subcore_axis_name="s")
    row_t = jax.ShapeDtypeStruct((BATCH, DIM), jnp.float32)

    @functools.partial(
        pl.kernel,
        out_type=(row_t, row_t, row_t, row_t),
        mesh=mesh,
        compiler_params=pltpu.CompilerParams(use_tc_tiling_on_sc=False),
        scratch_types=[
            pltpu.VMEM((NCH, CH), jnp.int32),
            pltpu.VMEM((NCH, CH), jnp.int32),
            pltpu.VMEM((B_PER_W, DIM), jnp.float32),
            pltpu.VMEM((B_PER_W, DIM), jnp.float32),
            pltpu.SemaphoreType.DMA,
            pltpu.SemaphoreType.DMA,
            pltpu.SemaphoreType.DMA,
            pltpu.SemaphoreType.DMA,
        ],
    )
    def k(u_hbm, i_hbm, mfu_hbm, mfi_hbm, mlu_hbm, mli_hbm,
          o_mfu, o_mfi, o_mlu, o_mli,
          uidx, iidx, rows_a, rows_b, sem_a, sem_b, sem_wa, sem_wb):
        wid = lax.axis_index("s") * NC + lax.axis_index("c")
        base = wid * B_PER_W

        pltpu.sync_copy(u_hbm.at[pl.ds(wid * NCH, NCH)], uidx)
        pltpu.sync_copy(i_hbm.at[pl.ds(wid * NCH, NCH)], iidx)

        def gather(table, idx, rows, sem):
            cps = []
            for c in range(NCH):
                cps.append(pltpu.async_copy(
                    table.at[idx.at[c]], rows.at[pl.ds(c * CH, CH)], sem))
            return cps

        def drain(cps):
            for cp in cps:
                cp.wait()

        ga = gather(mfu_hbm, uidx, rows_a, sem_a)
        gb = gather(mfi_hbm, iidx, rows_b, sem_b)
        drain(ga)
        wa = pltpu.async_copy(rows_a, o_mfu.at[pl.ds(base, B_PER_W)], sem_wa)
        drain(gb)
        wb = pltpu.async_copy(rows_b, o_mfi.at[pl.ds(base, B_PER_W)], sem_wb)

        wa.wait()
        ga = gather(mlu_hbm, uidx, rows_a, sem_a)
        wb.wait()
        gb = gather(mli_hbm, iidx, rows_b, sem_b)
        drain(ga)
        wa = pltpu.async_copy(rows_a, o_mlu.at[pl.ds(base, B_PER_W)], sem_wa)
        drain(gb)
        wb = pltpu.async_copy(rows_b, o_mli.at[pl.ds(base, B_PER_W)], sem_wb)
        wa.wait()
        wb.wait()

    return k(u2, i2, t_mfu, t_mfi, t_mlu, t_mli)


def _mm(a, b):
    return lax.dot_general(a, b, (((1,), (0,)), ((), ())),
                           precision=lax.Precision.HIGHEST,
                           preferred_element_type=jnp.float32)


BB = 2048


def _dense_body(xmfu, xmfi, xmlpu, xmlpi, w0u, w0i, b0, w1, b1, w2, b2,
                wfm, wfh, bf, o):
    h = _mm(xmlpu[...], w0u[...]) + _mm(xmlpi[...], w0i[...]) + b0[...]
    h = jnp.maximum(h, 0.0)
    h = jnp.maximum(_mm(h, w1[...]) + b1[...], 0.0)
    h = jnp.maximum(_mm(h, w2[...]) + b2[...], 0.0)
    xmf = xmfu[...] * xmfi[...]
    o[...] = _mm(xmf, wfm[...]) + _mm(h, wfh[...]) + bf[...]


def _tc_dense(xmfu, xmfi, xmlpu, xmlpi, w0u, w0i, b0, w1, b1, w2, b2,
              wfm, wfh, bf):
    bspec = lambda shape: pl.BlockSpec(shape, lambda i: (i, 0))
    wspec = lambda shape: pl.BlockSpec(shape, lambda i: (0, 0))
    return pl.pallas_call(
        _dense_body,
        grid=(BATCH // BB,),
        in_specs=[
            bspec((BB, DIM)), bspec((BB, DIM)),
            bspec((BB, DIM)), bspec((BB, DIM)),
            wspec((DIM, 64)), wspec((DIM, 64)), wspec((1, 64)),
            wspec((64, 32)), wspec((1, 32)),
            wspec((32, 16)), wspec((1, 16)),
            wspec((DIM, 1)), wspec((16, 1)), wspec((1, 1)),
        ],
        out_specs=pl.BlockSpec((BB, 1), lambda i: (i, 0)),
        out_shape=jax.ShapeDtypeStruct((BATCH, 1), jnp.float32),
    )(xmfu, xmfi, xmlpu, xmlpi, w0u, w0i, b0, w1, b1, w2, b2, wfm, wfh, bf)


def kernel(user, item, mf_user_embed, mf_item_embed, mlp_user_embed,
           mlp_item_embed, W0, b0, W1, b1, W2, b2, Wf, bf):
    u2 = user.astype(jnp.int32).reshape(BATCH // CH, CH)
    i2 = item.astype(jnp.int32).reshape(BATCH // CH, CH)
    xmfu, xmfi, xmlpu, xmlpi = _sc_gather4(
        u2, i2, mf_user_embed, mf_item_embed, mlp_user_embed, mlp_item_embed)

    w0t = W0.T
    w0u, w0i = w0t[:DIM], w0t[DIM:]
    wft = Wf.T
    wfm, wfh = wft[:DIM], wft[DIM:]
    out = _tc_dense(
        xmfu, xmfi, xmlpu, xmlpi,
        w0u, w0i, b0.reshape(1, -1),
        W1.T, b1.reshape(1, -1),
        W2.T, b2.reshape(1, -1),
        wfm, wfh, bf.reshape(1, 1))
    return out


# speedup vs baseline: 1.1918x; 1.1918x over previous
import functools

import jax
import jax.numpy as jnp
from jax import lax
from jax.experimental import pallas as pl
from jax.experimental.pallas import tpu as pltpu
from jax.experimental.pallas import tpu_sc as plsc

BATCH = 16384
DIM = 64
NC, NS = 2, 16
NW = NC * NS
B_PER_W = BATCH // NW
CH = 128
NCH = B_PER_W // CH


TW = 1024


def _xpose_body(x1_ref, x2_ref, o_ref):
    o_ref[:, :DIM] = x1_ref[...].T
    o_ref[:, DIM:] = x2_ref[...].T


def _tc_transpose_pack(ta_T, tb_T):
    n = ta_T.shape[1]
    return pl.pallas_call(
        _xpose_body,
        grid=(pl.cdiv(n, TW),),
        in_specs=[
            pl.BlockSpec((DIM, TW), lambda i: (0, i)),
            pl.BlockSpec((DIM, TW), lambda i: (0, i)),
        ],
        out_specs=pl.BlockSpec((TW, 2 * DIM), lambda i: (i, 0)),
        out_shape=jax.ShapeDtypeStruct((n, 2 * DIM), jnp.float32),
    )(ta_T, tb_T)


def _sc_gather2(u2, i2, p_user, p_item):
    mesh = plsc.VectorSubcoreMesh(core_axis_name="c", subcore_axis_name="s")
    row_t = jax.ShapeDtypeStruct((BATCH, 2 * DIM), jnp.float32)

    @functools.partial(
        pl.kernel,
        out_type=(row_t, row_t),
        mesh=mesh,
        compiler_params=pltpu.CompilerParams(use_tc_tiling_on_sc=False),
        scratch_types=[
            pltpu.VMEM((NCH, CH), jnp.int32),
            pltpu.VMEM((NCH, CH), jnp.int32),
            pltpu.VMEM((CH, 2 * DIM), jnp.float32),
            pltpu.VMEM((CH, 2 * DIM), jnp.float32),
            pltpu.SemaphoreType.DMA,
            pltpu.SemaphoreType.DMA,
            pltpu.SemaphoreType.DMA,
            pltpu.SemaphoreType.DMA,
        ],
    )
    def k(u_hbm, i_hbm, pu_hbm, pi_hbm, o_u, o_i,
          uidx, iidx, rows_a, rows_b, sem_a, sem_b, sem_wa, sem_wb):
        wid = lax.axis_index("s") * NC + lax.axis_index("c")
        base = wid * B_PER_W

        pltpu.sync_copy(u_hbm.at[pl.ds(wid * NCH, NCH)], uidx)
        pltpu.sync_copy(i_hbm.at[pl.ds(wid * NCH, NCH)], iidx)

        work = []
        for table, idx, out in ((pu_hbm, uidx, o_u), (pi_hbm, iidx, o_i)):
            for c in range(NCH):
                work.append((table, idx, c, out))

        bufs = ((rows_a, sem_a, sem_wa), (rows_b, sem_b, sem_wb))
        pending_w = [None, None]
        for n, (table, idx, c, out) in enumerate(work):
            rows, sem_g, sem_w = bufs[n % 2]
            if pending_w[n % 2] is not None:
                pending_w[n % 2].wait()
            g = pltpu.async_copy(table.at[idx.at[c]], rows, sem_g)
            g.wait()
            pending_w[n % 2] = pltpu.async_copy(
                rows, out.at[pl.ds(base + c * CH, CH)], sem_w)
        for w in pending_w:
            if w is not None:
                w.wait()

    return k(u2, i2, p_user, p_item)


def _mm(a, b):
    return lax.dot_general(a, b, (((1,), (0,)), ((), ())),
                           precision=lax.Precision.HIGHEST,
                           preferred_element_type=jnp.float32)


BB = 2048


def _dense_body(gu, gi, w0u, w0i, b0, w1, b1, w2, b2, wfm, wfh, bf, o):
    gub = gu[...]
    gib = gi[...]
    xmfu = gub[:, :DIM]
    xmlu = gub[:, DIM:]
    xmfi = gib[:, :DIM]
    xmli = gib[:, DIM:]

    h = _mm(xmlu, w0u[...]) + _mm(xmli, w0i[...]) + b0[...]
    h = jnp.maximum(h, 0.0)
    h = jnp.maximum(_mm(h, w1[...]) + b1[...], 0.0)
    h = jnp.maximum(_mm(h, w2[...]) + b2[...], 0.0)
    xmf = xmfu * xmfi
    o[...] = _mm(xmf, wfm[...]) + _mm(h, wfh[...]) + bf[...]


def _tc_dense(gu, gi, w0u, w0i, b0, w1, b1, w2, b2, wfm, wfh, bf):
    bspec = lambda shape: pl.BlockSpec(shape, lambda i: (i, 0))
    wspec = lambda shape: pl.BlockSpec(shape, lambda i: (0, 0))
    return pl.pallas_call(
        _dense_body,
        grid=(BATCH // BB,),
        in_specs=[
            bspec((BB, 2 * DIM)), bspec((BB, 2 * DIM)),
            wspec((DIM, 64)), wspec((DIM, 64)), wspec((1, 64)),
            wspec((64, 32)), wspec((1, 32)),
            wspec((32, 16)), wspec((1, 16)),
            wspec((DIM, 1)), wspec((16, 1)), wspec((1, 1)),
        ],
        out_specs=pl.BlockSpec((BB, 1), lambda i: (i, 0)),
        out_shape=jax.ShapeDtypeStruct((BATCH, 1), jnp.float32),
    )(gu, gi, w0u, w0i, b0, w1, b1, w2, b2, wfm, wfh, bf)


def kernel(user, item, mf_user_embed, mf_item_embed, mlp_user_embed,
           mlp_item_embed, W0, b0, W1, b1, W2, b2, Wf, bf):
    user = user.astype(jnp.int32)
    item = item.astype(jnp.int32)

    p_user = _tc_transpose_pack(mf_user_embed.T, mlp_user_embed.T)
    p_item = _tc_transpose_pack(mf_item_embed.T, mlp_item_embed.T)

    u2 = user.reshape(BATCH // CH, CH)
    i2 = item.reshape(BATCH // CH, CH)
    gu, gi = _sc_gather2(u2, i2, p_user, p_item)

    w0t = W0.T
    wft = Wf.T
    out = _tc_dense(
        gu, gi,
        w0t[:DIM], w0t[DIM:], b0.reshape(1, -1),
        W1.T, b1.reshape(1, -1),
        W2.T, b2.reshape(1, -1),
        wft[:DIM], wft[DIM:], bf.reshape(1, 1))
    return out


# speedup vs baseline: 2.0951x; 1.7579x over previous
import functools

import jax
import jax.numpy as jnp
from jax import lax
from jax.experimental import pallas as pl
from jax.experimental.pallas import tpu as pltpu
from jax.experimental.pallas import tpu_sc as plsc

BATCH = 16384
DIM = 64
NC, NS = 2, 16
NW = NC * NS
B_PER_W = BATCH // NW
CH = 128
NCH = B_PER_W // CH


TW = 4096


def _xpose_body(x1_ref, x2_ref, o_ref):
    o_ref[:, :DIM] = x1_ref[...].T
    o_ref[:, DIM:] = x2_ref[...].T


def _tc_transpose_pack(ta_T, tb_T):
    n = ta_T.shape[1]
    return pl.pallas_call(
        _xpose_body,
        grid=(pl.cdiv(n, TW),),
        in_specs=[
            pl.BlockSpec((DIM, TW), lambda i: (0, i)),
            pl.BlockSpec((DIM, TW), lambda i: (0, i)),
        ],
        out_specs=pl.BlockSpec((TW, 2 * DIM), lambda i: (i, 0)),
        out_shape=jax.ShapeDtypeStruct((n, 2 * DIM), jnp.float32),
    )(ta_T, tb_T)


def _sc_gather2(u2, i2, p_user, p_item):
    mesh = plsc.VectorSubcoreMesh(core_axis_name="c", subcore_axis_name="s")
    row_t = jax.ShapeDtypeStruct((BATCH, 2 * DIM), jnp.float32)

    @functools.partial(
        pl.kernel,
        out_type=(row_t, row_t),
        mesh=mesh,
        compiler_params=pltpu.CompilerParams(use_tc_tiling_on_sc=False),
        scratch_types=[
            pltpu.VMEM((NCH, CH), jnp.int32),
            pltpu.VMEM((NCH, CH), jnp.int32),
            pltpu.VMEM((CH, 2 * DIM), jnp.float32),
            pltpu.VMEM((CH, 2 * DIM), jnp.float32),
            pltpu.SemaphoreType.DMA,
            pltpu.SemaphoreType.DMA,
            pltpu.SemaphoreType.DMA,
            pltpu.SemaphoreType.DMA,
        ],
    )
    def k(u_hbm, i_hbm, pu_hbm, pi_hbm, o_u, o_i,
          uidx, iidx, rows_a, rows_b, sem_a, sem_b, sem_wa, sem_wb):
        wid = lax.axis_index("s") * NC + lax.axis_index("c")
        base = wid * B_PER_W

        pltpu.sync_copy(u_hbm.at[pl.ds(wid * NCH, NCH)], uidx)
        pltpu.sync_copy(i_hbm.at[pl.ds(wid * NCH, NCH)], iidx)

        work = []
        for table, idx, out in ((pu_hbm, uidx, o_u), (pi_hbm, iidx, o_i)):
            for c in range(NCH):
                work.append((table, idx, c, out))

        bufs = ((rows_a, sem_a, sem_wa), (rows_b, sem_b, sem_wb))
        pending_w = [None, None]
        for n, (table, idx, c, out) in enumerate(work):
            rows, sem_g, sem_w = bufs[n % 2]
            if pending_w[n % 2] is not None:
                pending_w[n % 2].wait()
            g = pltpu.async_copy(table.at[idx.at[c]], rows, sem_g)
            g.wait()
            pending_w[n % 2] = pltpu.async_copy(
                rows, out.at[pl.ds(base + c * CH, CH)], sem_w)
        for w in pending_w:
            if w is not None:
                w.wait()

    return k(u2, i2, p_user, p_item)


def _mm(a, b):
    return lax.dot_general(a, b, (((1,), (0,)), ((), ())),
                           preferred_element_type=jnp.float32)


BB = 2048


def _dense_body(gu, gi, w0u, w0i, b0, w1, b1, w2, b2, wfm, wfh, bf, o):
    gub = gu[...]
    gib = gi[...]
    xmfu = gub[:, :DIM]
    xmlu = gub[:, DIM:]
    xmfi = gib[:, :DIM]
    xmli = gib[:, DIM:]

    h = _mm(xmlu, w0u[...]) + _mm(xmli, w0i[...]) + b0[...]
    h = jnp.maximum(h, 0.0)
    h = jnp.maximum(_mm(h, w1[...]) + b1[...], 0.0)
    h = jnp.maximum(_mm(h, w2[...]) + b2[...], 0.0)
    xmf = xmfu * xmfi
    o[...] = _mm(xmf, wfm[...]) + _mm(h, wfh[...]) + bf[...]


def _tc_dense(gu, gi, w0u, w0i, b0, w1, b1, w2, b2, wfm, wfh, bf):
    bspec = lambda shape: pl.BlockSpec(shape, lambda i: (i, 0))
    wspec = lambda shape: pl.BlockSpec(shape, lambda i: (0, 0))
    return pl.pallas_call(
        _dense_body,
        grid=(BATCH // BB,),
        in_specs=[
            bspec((BB, 2 * DIM)), bspec((BB, 2 * DIM)),
            wspec((DIM, 64)), wspec((DIM, 64)), wspec((1, 64)),
            wspec((64, 32)), wspec((1, 32)),
            wspec((32, 16)), wspec((1, 16)),
            wspec((DIM, 1)), wspec((16, 1)), wspec((1, 1)),
        ],
        out_specs=pl.BlockSpec((BB, 1), lambda i: (i, 0)),
        out_shape=jax.ShapeDtypeStruct((BATCH, 1), jnp.float32),
    )(gu, gi, w0u, w0i, b0, w1, b1, w2, b2, wfm, wfh, bf)


def kernel(user, item, mf_user_embed, mf_item_embed, mlp_user_embed,
           mlp_item_embed, W0, b0, W1, b1, W2, b2, Wf, bf):
    user = user.astype(jnp.int32)
    item = item.astype(jnp.int32)

    p_user = _tc_transpose_pack(mf_user_embed.T, mlp_user_embed.T)
    p_item = _tc_transpose_pack(mf_item_embed.T, mlp_item_embed.T)

    u2 = user.reshape(BATCH // CH, CH)
    i2 = item.reshape(BATCH // CH, CH)
    gu, gi = _sc_gather2(u2, i2, p_user, p_item)

    w0t = W0.T
    wft = Wf.T
    out = _tc_dense(
        gu, gi,
        w0t[:DIM], w0t[DIM:], b0.reshape(1, -1),
        W1.T, b1.reshape(1, -1),
        W2.T, b2.reshape(1, -1),
        wft[:DIM], wft[DIM:], bf.reshape(1, 1))
    return out
